# Initial kernel scaffold; baseline (speedup 1.0000x reference)
import jax
import jax.numpy as jnp
from jax.experimental import pallas as pl


def kernel(x, Wg, W1, W2):
    raise NotImplementedError("write your pallas kernel here")


# speedup vs baseline: 5.4321x; 5.4321x over previous
import functools

import jax
import jax.numpy as jnp
from jax import lax
from jax.experimental import pallas as pl
from jax.experimental.pallas import tpu as pltpu
from jax.experimental.pallas import tpu_sc as plsc

B, T, DIM = 4, 8192, 768
E = 64
HIDDEN = 2048
NT = B * T
CAP = int(1.25 * NT / E) + 1
CAP_PAD = (CAP // 8 + 1) * 8
NSLOT = E * CAP_PAD
SENT = NT
TB = 256

NC, NS = 2, 16
NW = NC * NS

_mesh = plsc.VectorSubcoreMesh(core_axis_name="c", subcore_axis_name="s")
_sc_params = pltpu.CompilerParams(needs_layout_passes=False)


def _route_body(xb_ref, wg_ref, dsc_ref, dg_ref, w_ref, aux_ref, me_acc, cnt_acc):
    i = pl.program_id(0)
    n = pl.num_programs(0)

    @pl.when(i == 0)
    def _():
        me_acc[...] = jnp.zeros_like(me_acc)
        cnt_acc[...] = jnp.zeros_like(cnt_acc)

    xb = xb_ref[...]
    wg = wg_ref[...]
    logits = lax.dot_general(xb, wg, (((1,), (1,)), ((), ())),
                             preferred_element_type=jnp.float32)
    m = jnp.max(logits, axis=1, keepdims=True)
    p = jnp.exp(logits - m)
    s = jnp.sum(p, axis=1, keepdims=True)
    scores = p / s
    idx = jnp.argmax(logits, axis=1).astype(jnp.int32)
    w = jnp.max(p, axis=1) / s[:, 0]

    oh = (idx[:, None] == lax.broadcasted_iota(jnp.int32, (1, E), 1)
          ).astype(jnp.float32)
    ii = lax.broadcasted_iota(jnp.int32, (TB, TB), 0)
    jj = lax.broadcasted_iota(jnp.int32, (TB, TB), 1)
    tri = (ii > jj).astype(jnp.float32)
    excl = lax.dot_general(tri, oh, (((1,), (0,)), ((), ())),
                           preferred_element_type=jnp.float32)
    carry = cnt_acc[...]
    rank = jnp.sum(oh * (excl + carry), axis=1).astype(jnp.int32)
    cnt_acc[...] = carry + jnp.sum(oh, axis=0, keepdims=True)
    me_acc[...] = me_acc[...] + jnp.sum(scores, axis=0, keepdims=True)

    valid = rank < CAP
    slot = idx * CAP_PAD + rank
    dsc_ref[...] = jnp.where(valid, slot, NSLOT)
    dg_ref[...] = jnp.where(valid, slot, CAP)
    w_ref[...] = w

    @pl.when(i == n - 1)
    def _():
        me = me_acc[...] / NT
        assign = cnt_acc[...] / NT
        aux_ref[...] = (jnp.sum(me * assign) * (E * 0.01)).reshape(1, 1)


def _route(xf, Wg):
    return pl.pallas_call(
        _route_body,
        grid=(NT // TB,),
        in_specs=[
            pl.BlockSpec((TB, DIM), lambda i: (i, 0)),
            pl.BlockSpec((E, DIM), lambda i: (0, 0)),
        ],
        out_specs=[
            pl.BlockSpec((TB,), lambda i: (i,)),
            pl.BlockSpec((TB,), lambda i: (i,)),
            pl.BlockSpec((TB,), lambda i: (i,)),
            pl.BlockSpec((1, 1), lambda i: (0, 0)),
        ],
        out_shape=[
            jax.ShapeDtypeStruct((NT,), jnp.int32),
            jax.ShapeDtypeStruct((NT,), jnp.int32),
            jax.ShapeDtypeStruct((NT,), jnp.float32),
            jax.ShapeDtypeStruct((1, 1), jnp.float32),
        ],
        scratch_shapes=[
            pltpu.VMEM((1, E), jnp.float32),
            pltpu.VMEM((1, E), jnp.float32),
        ],
    )(xf, Wg)


_BCH = 2048


def _build_body(dsc_hbm, w_hbm, tok_out, wsl_out, tok_v, wsl_v, idx_st, w_st):
    wid = lax.axis_index("s") * NC + lax.axis_index("c")

    @pl.when(wid == 0)
    def _():
        def init_body(k, c):
            tok_v[pl.ds(k * 16, 16)] = jnp.full((16,), SENT, jnp.int32)
            wsl_v[pl.ds(k * 16, 16)] = jnp.zeros((16,), jnp.float32)
            return c

        lax.fori_loop(0, (NSLOT + 16) // 16, init_body, 0)

        for c in range(NT // _BCH):
            pltpu.sync_copy(dsc_hbm.at[pl.ds(c * _BCH, _BCH)], idx_st)
            pltpu.sync_copy(w_hbm.at[pl.ds(c * _BCH, _BCH)], w_st)

            def sc_body(j, carry, c=c):
                idxv = idx_st[pl.ds(j * 16, 16)]
                toks = (c * _BCH + j * 16
                        + lax.broadcasted_iota(jnp.int32, (16,), 0))
                plsc.store_scatter(tok_v, [idxv], toks)
                wv = w_st[pl.ds(j * 16, 16)]
                plsc.store_scatter(wsl_v, [idxv], wv)
                return carry

            lax.fori_loop(0, _BCH // 16, sc_body, 0)

        pltpu.sync_copy(tok_v, tok_out)
        pltpu.sync_copy(wsl_v, wsl_out)


def _build(dst_sc, w):
    return pl.kernel(
        _build_body,
        out_type=[
            jax.ShapeDtypeStruct((NSLOT + 16,), jnp.int32),
            jax.ShapeDtypeStruct((NSLOT + 16,), jnp.float32),
        ],
        mesh=_mesh,
        scratch_types=[
            pltpu.VMEM((NSLOT + 16,), jnp.int32),
            pltpu.VMEM((NSLOT + 16,), jnp.float32),
            pltpu.VMEM((_BCH,), jnp.int32),
            pltpu.VMEM((_BCH,), jnp.float32),
        ],
        compiler_params=_sc_params,
    )(dst_sc, w)


_GCH = 48


def _gather_x_body(tok_hbm, xpad_hbm, xe_out, idx_v, rows_v, sem):
    wid = lax.axis_index("s") * NC + lax.axis_index("c")
    per_w = NSLOT // NW

    def body(cc, carry):
        base = wid * per_w + cc * _GCH
        pltpu.sync_copy(tok_hbm.at[pl.ds(base, _GCH)], idx_v)
        pltpu.async_copy(xpad_hbm.at[idx_v], rows_v, sem).wait()
        pltpu.sync_copy(rows_v, xe_out.at[pl.ds(base, _GCH)])
        return carry

    lax.fori_loop(0, per_w // _GCH, body, 0)


def _gather_x(tok_slot, xpad):
    return pl.kernel(
        _gather_x_body,
        out_type=jax.ShapeDtypeStruct((NSLOT, DIM), jnp.float32),
        mesh=_mesh,
        scratch_types=[
            pltpu.VMEM((_GCH,), jnp.int32),
            pltpu.VMEM((_GCH, DIM), jnp.float32),
            pltpu.SemaphoreType.DMA,
        ],
        compiler_params=_sc_params,
    )(tok_slot, xpad)


def _ffn_body(xe_ref, w1_ref, w2_ref, ws_ref, ye_ref):
    xb = xe_ref[0].astype(jnp.bfloat16)
    h = lax.dot_general(xb, w1_ref[0], (((1,), (1,)), ((), ())),
                        preferred_element_type=jnp.float32)
    h = h * jax.nn.sigmoid(h)
    y = lax.dot_general(h.astype(jnp.bfloat16), w2_ref[0],
                        (((1,), (1,)), ((), ())),
                        preferred_element_type=jnp.float32)
    ws = ws_ref[0, 0, :].reshape(CAP_PAD, 1)
    ye_ref[0] = y * ws


def _ffn(xe3, W1b, W2b, wsl3):
    return pl.pallas_call(
        _ffn_body,
        grid=(E,),
        in_specs=[
            pl.BlockSpec((1, CAP_PAD, DIM), lambda e: (e, 0, 0)),
            pl.BlockSpec((1, HIDDEN, DIM), lambda e: (e, 0, 0)),
            pl.BlockSpec((1, DIM, HIDDEN), lambda e: (e, 0, 0)),
            pl.BlockSpec((1, 1, CAP_PAD), lambda e: (e, 0, 0)),
        ],
        out_specs=pl.BlockSpec((1, CAP_PAD, DIM), lambda e: (e, 0, 0)),
        out_shape=jax.ShapeDtypeStruct((E, CAP_PAD, DIM), jnp.float32),
    )(xe3, W1b, W2b, wsl3)


_OCH = 64


def _combine_body(dg_hbm, ye_hbm, out_hbm, idx_v, rows_v, sem):
    wid = lax.axis_index("s") * NC + lax.axis_index("c")
    per_w = NT // NW

    def body(cc, carry):
        base = wid * per_w + cc * _OCH
        pltpu.sync_copy(dg_hbm.at[pl.ds(base, _OCH)], idx_v)
        pltpu.async_copy(ye_hbm.at[idx_v], rows_v, sem).wait()
        pltpu.sync_copy(rows_v, out_hbm.at[pl.ds(base, _OCH)])
        return carry

    lax.fori_loop(0, per_w // _OCH, body, 0)


def _combine(dst_g, ye):
    return pl.kernel(
        _combine_body,
        out_type=jax.ShapeDtypeStruct((NT, DIM), jnp.float32),
        mesh=_mesh,
        scratch_types=[
            pltpu.VMEM((_OCH,), jnp.int32),
            pltpu.VMEM((_OCH, DIM), jnp.float32),
            pltpu.SemaphoreType.DMA,
        ],
        compiler_params=_sc_params,
    )(dst_g, ye)


def kernel(x, Wg, W1, W2):
    Bx, Tx, D = x.shape
    xf = x.reshape(Bx * Tx, D)
    dst_sc, dst_g, w, aux = _route(xf, Wg)
    tok_slot, w_slot = _build(dst_sc, w)
    xpad = jnp.concatenate([xf, jnp.zeros((1, D), xf.dtype)], axis=0)
    xe = _gather_x(tok_slot, xpad)
    ye = _ffn(
        xe.reshape(E, CAP_PAD, D),
        W1.astype(jnp.bfloat16),
        W2.astype(jnp.bfloat16),
        w_slot[:NSLOT].reshape(E, 1, CAP_PAD),
    )
    out = _combine(dst_g, ye.reshape(NSLOT, D))
    return out.reshape(Bx, Tx, D), aux[0, 0]


# speedup vs baseline: 7.0539x; 1.2985x over previous
import functools

import jax
import jax.numpy as jnp
from jax import lax
from jax.experimental import pallas as pl
from jax.experimental.pallas import tpu as pltpu
from jax.experimental.pallas import tpu_sc as plsc

B, T, DIM = 4, 8192, 768
E = 64
HIDDEN = 2048
NT = B * T
CAP = int(1.25 * NT / E) + 1
CAP_PAD = (CAP // 8 + 1) * 8
NSLOT = E * CAP_PAD
SENT = 0
TB = 256

NC, NS = 2, 16
NW = NC * NS

_mesh = plsc.VectorSubcoreMesh(core_axis_name="c", subcore_axis_name="s")
_sc_params = pltpu.CompilerParams(needs_layout_passes=False)


def _route_body(xb_ref, wg_ref, dsc_ref, dg_ref, w_ref, aux_ref, me_acc, cnt_acc):
    i = pl.program_id(0)
    n = pl.num_programs(0)

    @pl.when(i == 0)
    def _():
        me_acc[...] = jnp.zeros_like(me_acc)
        cnt_acc[...] = jnp.zeros_like(cnt_acc)

    xb = xb_ref[...]
    wg = wg_ref[...]
    logits = lax.dot_general(xb, wg, (((1,), (1,)), ((), ())),
                             preferred_element_type=jnp.float32)
    m = jnp.max(logits, axis=1, keepdims=True)
    p = jnp.exp(logits - m)
    s = jnp.sum(p, axis=1, keepdims=True)
    scores = p / s
    idx = jnp.argmax(logits, axis=1).astype(jnp.int32)
    w = jnp.max(p, axis=1) / s[:, 0]

    oh = (idx[:, None] == lax.broadcasted_iota(jnp.int32, (1, E), 1)
          ).astype(jnp.float32)
    ii = lax.broadcasted_iota(jnp.int32, (TB, TB), 0)
    jj = lax.broadcasted_iota(jnp.int32, (TB, TB), 1)
    tri = (ii > jj).astype(jnp.float32)
    excl = lax.dot_general(tri, oh, (((1,), (0,)), ((), ())),
                           preferred_element_type=jnp.float32)
    carry = cnt_acc[...]
    rank = jnp.sum(oh * (excl + carry), axis=1).astype(jnp.int32)
    cnt_acc[...] = carry + jnp.sum(oh, axis=0, keepdims=True)
    me_acc[...] = me_acc[...] + jnp.sum(scores, axis=0, keepdims=True)

    valid = rank < CAP
    slot = idx * CAP_PAD + rank
    dsc_ref[...] = jnp.where(valid, slot, NSLOT)
    dg_ref[...] = jnp.where(valid, slot, CAP)
    w_ref[...] = w

    @pl.when(i == n - 1)
    def _():
        me = me_acc[...] / NT
        assign = cnt_acc[...] / NT
        aux_ref[...] = (jnp.sum(me * assign) * (E * 0.01)).reshape(1, 1)


def _route(xf, Wg):
    return pl.pallas_call(
        _route_body,
        grid=(NT // TB,),
        in_specs=[
            pl.BlockSpec((TB, DIM), lambda i: (i, 0)),
            pl.BlockSpec((E, DIM), lambda i: (0, 0)),
        ],
        out_specs=[
            pl.BlockSpec((TB,), lambda i: (i,)),
            pl.BlockSpec((TB,), lambda i: (i,)),
            pl.BlockSpec((TB,), lambda i: (i,)),
            pl.BlockSpec((1, 1), lambda i: (0, 0)),
        ],
        out_shape=[
            jax.ShapeDtypeStruct((NT,), jnp.int32),
            jax.ShapeDtypeStruct((NT,), jnp.int32),
            jax.ShapeDtypeStruct((NT,), jnp.float32),
            jax.ShapeDtypeStruct((1, 1), jnp.float32),
        ],
        scratch_shapes=[
            pltpu.VMEM((1, E), jnp.float32),
            pltpu.VMEM((1, E), jnp.float32),
        ],
    )(xf, Wg)


_BCH = 2048


def _build_body(dsc_hbm, w_hbm, tok_out, wsl_out, tok_v, wsl_v, idx_st, w_st):
    wid = lax.axis_index("s") * NC + lax.axis_index("c")

    @pl.when(wid == 0)
    def _():
        def init_body(k, c):
            tok_v[pl.ds(k * 16, 16)] = jnp.full((16,), SENT, jnp.int32)
            wsl_v[pl.ds(k * 16, 16)] = jnp.zeros((16,), jnp.float32)
            return c

        lax.fori_loop(0, (NSLOT + 16) // 16, init_body, 0)

        for c in range(NT // _BCH):
            pltpu.sync_copy(dsc_hbm.at[pl.ds(c * _BCH, _BCH)], idx_st)
            pltpu.sync_copy(w_hbm.at[pl.ds(c * _BCH, _BCH)], w_st)

            def sc_body(j, carry, c=c):
                idxv = idx_st[pl.ds(j * 16, 16)]
                toks = (c * _BCH + j * 16
                        + lax.broadcasted_iota(jnp.int32, (16,), 0))
                plsc.store_scatter(tok_v, [idxv], toks)
                wv = w_st[pl.ds(j * 16, 16)]
                plsc.store_scatter(wsl_v, [idxv], wv)
                return carry

            lax.fori_loop(0, _BCH // 16, sc_body, 0)

        pltpu.sync_copy(tok_v, tok_out)
        pltpu.sync_copy(wsl_v, wsl_out)


def _build(dst_sc, w):
    return pl.kernel(
        _build_body,
        out_type=[
            jax.ShapeDtypeStruct((NSLOT + 16,), jnp.int32),
            jax.ShapeDtypeStruct((NSLOT + 16,), jnp.float32),
        ],
        mesh=_mesh,
        scratch_types=[
            pltpu.VMEM((NSLOT + 16,), jnp.int32),
            pltpu.VMEM((NSLOT + 16,), jnp.float32),
            pltpu.VMEM((_BCH,), jnp.int32),
            pltpu.VMEM((_BCH,), jnp.float32),
        ],
        compiler_params=_sc_params,
    )(dst_sc, w)


_GCH = 48


def _gather_x_body(tok_hbm, xpad_hbm, xe_out, idx_v, rows_v, sem):
    wid = lax.axis_index("s") * NC + lax.axis_index("c")
    per_w = NSLOT // NW

    def body(cc, carry):
        base = wid * per_w + cc * _GCH
        pltpu.sync_copy(tok_hbm.at[pl.ds(base, _GCH)], idx_v)
        pltpu.async_copy(xpad_hbm.at[idx_v], rows_v, sem).wait()
        pltpu.sync_copy(rows_v, xe_out.at[pl.ds(base, _GCH)])
        return carry

    lax.fori_loop(0, per_w // _GCH, body, 0)


def _gather_x(tok_slot, xpad):
    return pl.kernel(
        _gather_x_body,
        out_type=jax.ShapeDtypeStruct((NSLOT, DIM), jnp.float32),
        mesh=_mesh,
        scratch_types=[
            pltpu.VMEM((_GCH,), jnp.int32),
            pltpu.VMEM((_GCH, DIM), jnp.float32),
            pltpu.SemaphoreType.DMA,
        ],
        compiler_params=_sc_params,
    )(tok_slot, xpad)


def _ffn_body(xe_ref, w1_ref, w2_ref, ws_ref, ye_ref):
    xb = xe_ref[0].astype(jnp.bfloat16)
    w1 = w1_ref[0].astype(jnp.bfloat16)
    h = lax.dot_general(xb, w1, (((1,), (1,)), ((), ())),
                        preferred_element_type=jnp.float32)
    h = h * jax.nn.sigmoid(h)
    w2 = w2_ref[0].astype(jnp.bfloat16)
    y = lax.dot_general(h.astype(jnp.bfloat16), w2,
                        (((1,), (1,)), ((), ())),
                        preferred_element_type=jnp.float32)
    ws = ws_ref[0, 0, :].reshape(CAP_PAD, 1)
    ye_ref[0] = y * ws


def _ffn(xe3, W1b, W2b, wsl3):
    return pl.pallas_call(
        _ffn_body,
        grid=(E,),
        in_specs=[
            pl.BlockSpec((1, CAP_PAD, DIM), lambda e: (e, 0, 0)),
            pl.BlockSpec((1, HIDDEN, DIM), lambda e: (e, 0, 0)),
            pl.BlockSpec((1, DIM, HIDDEN), lambda e: (e, 0, 0)),
            pl.BlockSpec((1, 1, CAP_PAD), lambda e: (e, 0, 0)),
        ],
        out_specs=pl.BlockSpec((1, CAP_PAD, DIM), lambda e: (e, 0, 0)),
        out_shape=jax.ShapeDtypeStruct((E, CAP_PAD, DIM), jnp.float32),
    )(xe3, W1b, W2b, wsl3)


_OCH = 64


def _combine_body(dg_hbm, ye_hbm, out_hbm, idx_v, rows_v, sem):
    wid = lax.axis_index("s") * NC + lax.axis_index("c")
    per_w = NT // NW

    def body(cc, carry):
        base = wid * per_w + cc * _OCH
        pltpu.sync_copy(dg_hbm.at[pl.ds(base, _OCH)], idx_v)
        pltpu.async_copy(ye_hbm.at[idx_v], rows_v, sem).wait()
        pltpu.sync_copy(rows_v, out_hbm.at[pl.ds(base, _OCH)])
        return carry

    lax.fori_loop(0, per_w // _OCH, body, 0)


def _combine(dst_g, ye):
    return pl.kernel(
        _combine_body,
        out_type=jax.ShapeDtypeStruct((NT, DIM), jnp.float32),
        mesh=_mesh,
        scratch_types=[
            pltpu.VMEM((_OCH,), jnp.int32),
            pltpu.VMEM((_OCH, DIM), jnp.float32),
            pltpu.SemaphoreType.DMA,
        ],
        compiler_params=_sc_params,
    )(dst_g, ye)


def kernel(x, Wg, W1, W2):
    Bx, Tx, D = x.shape
    xf = x.reshape(Bx * Tx, D)
    dst_sc, dst_g, w, aux = _route(xf, Wg)
    tok_slot, w_slot = _build(dst_sc, w)
    xe = _gather_x(tok_slot, xf)
    ye = _ffn(
        xe.reshape(E, CAP_PAD, D),
        W1,
        W2,
        w_slot[:NSLOT].reshape(E, 1, CAP_PAD),
    )
    out = _combine(dst_g, ye.reshape(NSLOT, D))
    return out.reshape(Bx, Tx, D), aux[0, 0]


# speedup vs baseline: 11.3462x; 1.6085x over previous
import functools

import jax
import jax.numpy as jnp
from jax import lax
from jax.experimental import pallas as pl
from jax.experimental.pallas import tpu as pltpu
from jax.experimental.pallas import tpu_sc as plsc

B, T, DIM = 4, 8192, 768
E = 64
HIDDEN = 2048
NT = B * T
CAP = int(1.25 * NT / E) + 1
CAP_PAD = (CAP // 8 + 1) * 8
NSLOT = E * CAP_PAD
TB = 512

NC, NS = 2, 16
NW = NC * NS

_mesh = plsc.VectorSubcoreMesh(core_axis_name="c", subcore_axis_name="s")
_sc_params = pltpu.CompilerParams(needs_layout_passes=False)


def _route_body(xb_ref, wg_ref, dsc_ref, dg_ref, w_ref, aux_ref, me_acc, cnt_acc):
    i = pl.program_id(0)
    n = pl.num_programs(0)

    @pl.when(i == 0)
    def _():
        me_acc[...] = jnp.zeros_like(me_acc)
        cnt_acc[...] = jnp.zeros_like(cnt_acc)

    xb = xb_ref[...]
    wg = wg_ref[...]
    logits = lax.dot_general(xb, wg, (((1,), (1,)), ((), ())),
                             preferred_element_type=jnp.float32)
    m = jnp.max(logits, axis=1, keepdims=True)
    p = jnp.exp(logits - m)
    s = jnp.sum(p, axis=1, keepdims=True)
    scores = p / s
    idx = jnp.argmax(logits, axis=1).astype(jnp.int32)
    w = jnp.max(p, axis=1) / s[:, 0]

    oh = (idx[:, None] == lax.broadcasted_iota(jnp.int32, (1, E), 1)
          ).astype(jnp.float32)
    ii = lax.broadcasted_iota(jnp.int32, (TB, TB), 0)
    jj = lax.broadcasted_iota(jnp.int32, (TB, TB), 1)
    tri = (ii > jj).astype(jnp.float32)
    excl = lax.dot_general(tri, oh, (((1,), (0,)), ((), ())),
                           preferred_element_type=jnp.float32)
    carry = cnt_acc[...]
    rank = jnp.sum(oh * (excl + carry), axis=1).astype(jnp.int32)
    cnt_acc[...] = carry + jnp.sum(oh, axis=0, keepdims=True)
    me_acc[...] = me_acc[...] + jnp.sum(scores, axis=0, keepdims=True)

    valid = rank < CAP
    slot = idx * CAP_PAD + rank
    dsc_ref[...] = jnp.where(valid, slot, NSLOT)
    dg_ref[...] = jnp.where(valid, slot, CAP)
    w_ref[...] = w

    @pl.when(i == n - 1)
    def _():
        me = me_acc[...] / NT
        assign = cnt_acc[...] / NT
        aux_ref[...] = (jnp.sum(me * assign) * (E * 0.01)).reshape(1, 1)


def _route(xf, Wg):
    return pl.pallas_call(
        _route_body,
        grid=(NT // TB,),
        in_specs=[
            pl.BlockSpec((TB, DIM), lambda i: (i, 0)),
            pl.BlockSpec((E, DIM), lambda i: (0, 0)),
        ],
        out_specs=[
            pl.BlockSpec((TB,), lambda i: (i,)),
            pl.BlockSpec((TB,), lambda i: (i,)),
            pl.BlockSpec((TB,), lambda i: (i,)),
            pl.BlockSpec((1, 1), lambda i: (0, 0)),
        ],
        out_shape=[
            jax.ShapeDtypeStruct((NT,), jnp.int32),
            jax.ShapeDtypeStruct((NT,), jnp.int32),
            jax.ShapeDtypeStruct((NT,), jnp.float32),
            jax.ShapeDtypeStruct((1, 1), jnp.float32),
        ],
        scratch_shapes=[
            pltpu.VMEM((1, E), jnp.float32),
            pltpu.VMEM((1, E), jnp.float32),
        ],
    )(xf, Wg)


_BCH = 2048


def _build_body(dsc_hbm, w_hbm, tok_out, wsl_out, tok_v, wsl_v, idx_st, w_st):
    wid = lax.axis_index("s") * NC + lax.axis_index("c")

    @pl.when(wid == 0)
    def _():
        def init_body(k, c):
            fill = (k * 16 + lax.broadcasted_iota(jnp.int32, (16,), 0)) & (NT - 1)
            tok_v[pl.ds(k * 16, 16)] = fill
            wsl_v[pl.ds(k * 16, 16)] = jnp.zeros((16,), jnp.float32)
            return c

        lax.fori_loop(0, (NSLOT + 16) // 16, init_body, 0)

        for c in range(NT // _BCH):
            pltpu.sync_copy(dsc_hbm.at[pl.ds(c * _BCH, _BCH)], idx_st)
            pltpu.sync_copy(w_hbm.at[pl.ds(c * _BCH, _BCH)], w_st)

            def sc_body(j, carry, c=c):
                idxv = idx_st[pl.ds(j * 16, 16)]
                toks = (c * _BCH + j * 16
                        + lax.broadcasted_iota(jnp.int32, (16,), 0))
                plsc.store_scatter(tok_v, [idxv], toks)
                wv = w_st[pl.ds(j * 16, 16)]
                plsc.store_scatter(wsl_v, [idxv], wv)
                return carry

            lax.fori_loop(0, _BCH // 16, sc_body, 0)

        pltpu.sync_copy(tok_v, tok_out)
        pltpu.sync_copy(wsl_v, wsl_out)


def _build(dst_sc, w):
    return pl.kernel(
        _build_body,
        out_type=[
            jax.ShapeDtypeStruct((NSLOT + 16,), jnp.int32),
            jax.ShapeDtypeStruct((NSLOT + 16,), jnp.float32),
        ],
        mesh=_mesh,
        scratch_types=[
            pltpu.VMEM((NSLOT + 16,), jnp.int32),
            pltpu.VMEM((NSLOT + 16,), jnp.float32),
            pltpu.VMEM((_BCH,), jnp.int32),
            pltpu.VMEM((_BCH,), jnp.float32),
        ],
        compiler_params=_sc_params,
    )(dst_sc, w)


_GCH = 48


def _gather_x_body(tok_hbm, xpad_hbm, xe_out, idx_v, rows_v, sem):
    wid = lax.axis_index("s") * NC + lax.axis_index("c")
    per_w = NSLOT // NW

    def body(cc, carry):
        base = wid * per_w + cc * _GCH
        pltpu.sync_copy(tok_hbm.at[pl.ds(base, _GCH)], idx_v)
        pltpu.async_copy(xpad_hbm.at[idx_v], rows_v, sem).wait()
        pltpu.sync_copy(rows_v, xe_out.at[pl.ds(base, _GCH)])
        return carry

    lax.fori_loop(0, per_w // _GCH, body, 0)


def _gather_x(tok_slot, xpad):
    return pl.kernel(
        _gather_x_body,
        out_type=jax.ShapeDtypeStruct((NSLOT, DIM), jnp.float32),
        mesh=_mesh,
        scratch_types=[
            pltpu.VMEM((_GCH,), jnp.int32),
            pltpu.VMEM((_GCH, DIM), jnp.float32),
            pltpu.SemaphoreType.DMA,
        ],
        compiler_params=_sc_params,
    )(tok_slot, xpad)


def _ffn_body(xe_ref, w1_ref, w2_ref, ws_ref, ye_ref):
    xb = xe_ref[0].astype(jnp.bfloat16)
    w1 = w1_ref[0].astype(jnp.bfloat16)
    h = lax.dot_general(xb, w1, (((1,), (1,)), ((), ())),
                        preferred_element_type=jnp.float32)
    h = h * jax.nn.sigmoid(h)
    w2 = w2_ref[0].astype(jnp.bfloat16)
    y = lax.dot_general(h.astype(jnp.bfloat16), w2,
                        (((1,), (1,)), ((), ())),
                        preferred_element_type=jnp.float32)
    ws = ws_ref[0, 0, :].reshape(CAP_PAD, 1)
    ye_ref[0] = y * ws


def _ffn(xe3, W1b, W2b, wsl3):
    return pl.pallas_call(
        _ffn_body,
        grid=(E,),
        in_specs=[
            pl.BlockSpec((1, CAP_PAD, DIM), lambda e: (e, 0, 0)),
            pl.BlockSpec((1, HIDDEN, DIM), lambda e: (e, 0, 0)),
            pl.BlockSpec((1, DIM, HIDDEN), lambda e: (e, 0, 0)),
            pl.BlockSpec((1, 1, CAP_PAD), lambda e: (e, 0, 0)),
        ],
        out_specs=pl.BlockSpec((1, CAP_PAD, DIM), lambda e: (e, 0, 0)),
        out_shape=jax.ShapeDtypeStruct((E, CAP_PAD, DIM), jnp.float32),
    )(xe3, W1b, W2b, wsl3)


_OCH = 64


def _combine_body(dg_hbm, ye_hbm, out_hbm, idx_v, rows_v, sem):
    wid = lax.axis_index("s") * NC + lax.axis_index("c")
    per_w = NT // NW

    def body(cc, carry):
        base = wid * per_w + cc * _OCH
        pltpu.sync_copy(dg_hbm.at[pl.ds(base, _OCH)], idx_v)
        pltpu.async_copy(ye_hbm.at[idx_v], rows_v, sem).wait()
        pltpu.sync_copy(rows_v, out_hbm.at[pl.ds(base, _OCH)])
        return carry

    lax.fori_loop(0, per_w // _OCH, body, 0)


def _combine(dst_g, ye):
    return pl.kernel(
        _combine_body,
        out_type=jax.ShapeDtypeStruct((NT, DIM), jnp.float32),
        mesh=_mesh,
        scratch_types=[
            pltpu.VMEM((_OCH,), jnp.int32),
            pltpu.VMEM((_OCH, DIM), jnp.float32),
            pltpu.SemaphoreType.DMA,
        ],
        compiler_params=_sc_params,
    )(dst_g, ye)


def kernel(x, Wg, W1, W2):
    Bx, Tx, D = x.shape
    xf = x.reshape(Bx * Tx, D)
    dst_sc, dst_g, w, aux = _route(xf, Wg)
    tok_slot, w_slot = _build(dst_sc, w)
    xe = _gather_x(tok_slot, xf)
    ye = _ffn(
        xe.reshape(E, CAP_PAD, D),
        W1,
        W2,
        w_slot[:NSLOT].reshape(E, 1, CAP_PAD),
    )
    out = _combine(dst_g, ye.reshape(NSLOT, D))
    return out.reshape(Bx, Tx, D), aux[0, 0]


# speedup vs baseline: 11.7031x; 1.0315x over previous
import functools

import jax
import jax.numpy as jnp
from jax import lax
from jax.experimental import pallas as pl
from jax.experimental.pallas import tpu as pltpu
from jax.experimental.pallas import tpu_sc as plsc

B, T, DIM = 4, 8192, 768
E = 64
HIDDEN = 2048
NT = B * T
CAP = int(1.25 * NT / E) + 1
CAP_PAD = (CAP // 8 + 1) * 8
NSLOT = E * CAP_PAD
TB = 512

NC, NS = 2, 16
NW = NC * NS

_mesh = plsc.VectorSubcoreMesh(core_axis_name="c", subcore_axis_name="s")
_sc_params = pltpu.CompilerParams(needs_layout_passes=False)


def _route_body(xb_ref, wg_ref, dsc_ref, dg_ref, w_ref, aux_ref, me_acc, cnt_acc,
                tri_v):
    i = pl.program_id(0)
    n = pl.num_programs(0)

    @pl.when(i == 0)
    def _():
        me_acc[...] = jnp.zeros_like(me_acc)
        cnt_acc[...] = jnp.zeros_like(cnt_acc)
        ii = lax.broadcasted_iota(jnp.int32, (TB, TB), 0)
        jj = lax.broadcasted_iota(jnp.int32, (TB, TB), 1)
        tri_v[...] = (ii > jj).astype(jnp.float32)

    xb = xb_ref[...]
    wg = wg_ref[...]
    logits = lax.dot_general(xb, wg, (((1,), (1,)), ((), ())),
                             preferred_element_type=jnp.float32)
    m = jnp.max(logits, axis=1, keepdims=True)
    p = jnp.exp(logits - m)
    s = jnp.sum(p, axis=1, keepdims=True)
    scores = p / s
    idx = jnp.argmax(logits, axis=1).astype(jnp.int32)
    w = jnp.max(p, axis=1) / s[:, 0]

    oh = (idx[:, None] == lax.broadcasted_iota(jnp.int32, (1, E), 1)
          ).astype(jnp.float32)
    excl = lax.dot_general(tri_v[...], oh, (((1,), (0,)), ((), ())),
                           preferred_element_type=jnp.float32)
    carry = cnt_acc[...]
    rank = jnp.sum(oh * (excl + carry), axis=1).astype(jnp.int32)
    cnt_acc[...] = carry + jnp.sum(oh, axis=0, keepdims=True)
    me_acc[...] = me_acc[...] + jnp.sum(scores, axis=0, keepdims=True)

    valid = rank < CAP
    slot = idx * CAP_PAD + rank
    dsc_ref[...] = jnp.where(valid, slot, NSLOT)
    dg_ref[...] = jnp.where(valid, slot, CAP)
    w_ref[...] = w

    @pl.when(i == n - 1)
    def _():
        me = me_acc[...] / NT
        assign = cnt_acc[...] / NT
        aux_ref[...] = (jnp.sum(me * assign) * (E * 0.01)).reshape(1, 1)


def _route(xf, Wg):
    return pl.pallas_call(
        _route_body,
        grid=(NT // TB,),
        in_specs=[
            pl.BlockSpec((TB, DIM), lambda i: (i, 0)),
            pl.BlockSpec((E, DIM), lambda i: (0, 0)),
        ],
        out_specs=[
            pl.BlockSpec((TB,), lambda i: (i,)),
            pl.BlockSpec((TB,), lambda i: (i,)),
            pl.BlockSpec((TB,), lambda i: (i,)),
            pl.BlockSpec((1, 1), lambda i: (0, 0)),
        ],
        out_shape=[
            jax.ShapeDtypeStruct((NT,), jnp.int32),
            jax.ShapeDtypeStruct((NT,), jnp.int32),
            jax.ShapeDtypeStruct((NT,), jnp.float32),
            jax.ShapeDtypeStruct((1, 1), jnp.float32),
        ],
        scratch_shapes=[
            pltpu.VMEM((1, E), jnp.float32),
            pltpu.VMEM((1, E), jnp.float32),
            pltpu.VMEM((TB, TB), jnp.float32),
        ],
    )(xf, Wg)


_BCH = 2048


def _build_body(dsc_hbm, w_hbm, tok_out, wsl_out, tok_v, wsl_v, idx_st, w_st):
    wid = lax.axis_index("s") * NC + lax.axis_index("c")

    @pl.when(wid == 0)
    def _():
        def init_body(k, c):
            fill = (k * 16 + lax.broadcasted_iota(jnp.int32, (16,), 0)) & (NT - 1)
            tok_v[pl.ds(k * 16, 16)] = fill
            wsl_v[pl.ds(k * 16, 16)] = jnp.zeros((16,), jnp.float32)
            return c

        lax.fori_loop(0, (NSLOT + 16) // 16, init_body, 0)

        for c in range(NT // _BCH):
            pltpu.sync_copy(dsc_hbm.at[pl.ds(c * _BCH, _BCH)], idx_st)
            pltpu.sync_copy(w_hbm.at[pl.ds(c * _BCH, _BCH)], w_st)

            def sc_body(j, carry, c=c):
                idxv = idx_st[pl.ds(j * 16, 16)]
                toks = (c * _BCH + j * 16
                        + lax.broadcasted_iota(jnp.int32, (16,), 0))
                plsc.store_scatter(tok_v, [idxv], toks)
                wv = w_st[pl.ds(j * 16, 16)]
                plsc.store_scatter(wsl_v, [idxv], wv)
                return carry

            lax.fori_loop(0, _BCH // 16, sc_body, 0)

        pltpu.sync_copy(tok_v, tok_out)
        pltpu.sync_copy(wsl_v, wsl_out)


def _build(dst_sc, w):
    return pl.kernel(
        _build_body,
        out_type=[
            jax.ShapeDtypeStruct((NSLOT + 16,), jnp.int32),
            jax.ShapeDtypeStruct((NSLOT + 16,), jnp.float32),
        ],
        mesh=_mesh,
        scratch_types=[
            pltpu.VMEM((NSLOT + 16,), jnp.int32),
            pltpu.VMEM((NSLOT + 16,), jnp.float32),
            pltpu.VMEM((_BCH,), jnp.int32),
            pltpu.VMEM((_BCH,), jnp.float32),
        ],
        compiler_params=_sc_params,
    )(dst_sc, w)


NG = 2
E_G = E // NG
NSLOT_G = NSLOT // NG
_GCH = 72


def _gather_x_body(tok_hbm, xf_hbm, xe_out, idx_v, rows_v, sem, *, group):
    wid = lax.axis_index("s") * NC + lax.axis_index("c")
    per_w = NSLOT_G // NW

    def body(cc, carry):
        base = wid * per_w + cc * _GCH
        pltpu.sync_copy(tok_hbm.at[pl.ds(group * NSLOT_G + base, _GCH)], idx_v)
        pltpu.async_copy(xf_hbm.at[idx_v], rows_v, sem).wait()
        pltpu.sync_copy(rows_v, xe_out.at[pl.ds(base, _GCH)])
        return carry

    lax.fori_loop(0, per_w // _GCH, body, 0)


def _gather_x(tok_slot, xf, group):
    return pl.kernel(
        functools.partial(_gather_x_body, group=group),
        out_type=jax.ShapeDtypeStruct((NSLOT_G, DIM), jnp.float32),
        mesh=_mesh,
        scratch_types=[
            pltpu.VMEM((_GCH,), jnp.int32),
            pltpu.VMEM((_GCH, DIM), jnp.float32),
            pltpu.SemaphoreType.DMA,
        ],
        compiler_params=_sc_params,
    )(tok_slot, xf)


def _ffn_body(xe_ref, w1_ref, w2_ref, ws_ref, ye_ref):
    xb = xe_ref[0].astype(jnp.bfloat16)
    w1 = w1_ref[0].astype(jnp.bfloat16)
    h = lax.dot_general(xb, w1, (((1,), (1,)), ((), ())),
                        preferred_element_type=jnp.float32)
    h = h * jax.nn.sigmoid(h)
    w2 = w2_ref[0].astype(jnp.bfloat16)
    y = lax.dot_general(h.astype(jnp.bfloat16), w2,
                        (((1,), (1,)), ((), ())),
                        preferred_element_type=jnp.float32)
    ws = ws_ref[0, 0, :].reshape(CAP_PAD, 1)
    ye_ref[0] = y * ws


def _ffn_alias_body(ye_in_ref, xe_ref, w1_ref, w2_ref, ws_ref, ye_ref):
    del ye_in_ref
    _ffn_body(xe_ref, w1_ref, w2_ref, ws_ref, ye_ref)


def _ffn(xe3, W1, W2, wsl3, group, ye_prev):
    eoff = group * E_G
    specs = [
        pl.BlockSpec((1, CAP_PAD, DIM), lambda e: (e, 0, 0)),
        pl.BlockSpec((1, HIDDEN, DIM), lambda e, eoff=eoff: (e + eoff, 0, 0)),
        pl.BlockSpec((1, DIM, HIDDEN), lambda e, eoff=eoff: (e + eoff, 0, 0)),
        pl.BlockSpec((1, 1, CAP_PAD), lambda e, eoff=eoff: (e + eoff, 0, 0)),
    ]
    out_spec = pl.BlockSpec(
        (1, CAP_PAD, DIM), lambda e, eoff=eoff: (e + eoff, 0, 0))
    out_shape = jax.ShapeDtypeStruct((E, CAP_PAD, DIM), jnp.float32)
    if group == 0:
        return pl.pallas_call(
            _ffn_body,
            grid=(E_G,),
            in_specs=specs,
            out_specs=out_spec,
            out_shape=out_shape,
        )(xe3, W1, W2, wsl3)
    return pl.pallas_call(
        _ffn_alias_body,
        grid=(E_G,),
        in_specs=[pl.BlockSpec(memory_space=pl.ANY)] + specs,
        out_specs=out_spec,
        out_shape=out_shape,
        input_output_aliases={0: 0},
    )(ye_prev, xe3, W1, W2, wsl3)


_OCH = 64


def _combine_body(dg_hbm, ye_hbm, out_hbm, idx_v, rows_v, sem):
    wid = lax.axis_index("s") * NC + lax.axis_index("c")
    per_w = NT // NW

    def body(cc, carry):
        base = wid * per_w + cc * _OCH
        pltpu.sync_copy(dg_hbm.at[pl.ds(base, _OCH)], idx_v)
        pltpu.async_copy(ye_hbm.at[idx_v], rows_v, sem).wait()
        pltpu.sync_copy(rows_v, out_hbm.at[pl.ds(base, _OCH)])
        return carry

    lax.fori_loop(0, per_w // _OCH, body, 0)


def _combine(dst_g, ye):
    return pl.kernel(
        _combine_body,
        out_type=jax.ShapeDtypeStruct((NT, DIM), jnp.float32),
        mesh=_mesh,
        scratch_types=[
            pltpu.VMEM((_OCH,), jnp.int32),
            pltpu.VMEM((_OCH, DIM), jnp.float32),
            pltpu.SemaphoreType.DMA,
        ],
        compiler_params=_sc_params,
    )(dst_g, ye)


def kernel(x, Wg, W1, W2):
    Bx, Tx, D = x.shape
    xf = x.reshape(Bx * Tx, D)
    dst_sc, dst_g, w, aux = _route(xf, Wg)
    tok_slot, w_slot = _build(dst_sc, w)
    wsl3 = w_slot[:NSLOT].reshape(E, 1, CAP_PAD)
    ye = None
    for g in range(NG):
        xe_g = _gather_x(tok_slot, xf, g)
        ye = _ffn(xe_g.reshape(E_G, CAP_PAD, D), W1, W2, wsl3, g, ye)
    out = _combine(dst_g, ye.reshape(NSLOT, D))
    return out.reshape(Bx, Tx, D), aux[0, 0]


# speedup vs baseline: 13.1767x; 1.1259x over previous
import functools

import jax
import jax.numpy as jnp
from jax import lax
from jax.experimental import pallas as pl
from jax.experimental.pallas import tpu as pltpu
from jax.experimental.pallas import tpu_sc as plsc

B, T, DIM = 4, 8192, 768
E = 64
HIDDEN = 2048
NT = B * T
CAP = int(1.25 * NT / E) + 1
CAP_PAD = (CAP // 8 + 1) * 8
NSLOT = E * CAP_PAD
TB = 512

NC, NS = 2, 16
NW = NC * NS

_mesh = plsc.VectorSubcoreMesh(core_axis_name="c", subcore_axis_name="s")
_sc_params = pltpu.CompilerParams(needs_layout_passes=False)


PK = DIM // 2


def _rnd_hi(v):
    u = lax.bitcast_convert_type(v, jnp.uint32)
    u = u + jnp.uint32(0x7FFF) + ((u >> jnp.uint32(16)) & jnp.uint32(1))
    return u & jnp.uint32(0xFFFF0000)


def _route_body(xb_ref, wg_ref, dsc_ref, dg_ref, w_ref, xpk_ref, aux_ref,
                me_acc, cnt_acc, tri_v):
    i = pl.program_id(0)
    n = pl.num_programs(0)

    @pl.when(i == 0)
    def _():
        me_acc[...] = jnp.zeros_like(me_acc)
        cnt_acc[...] = jnp.zeros_like(cnt_acc)
        jj = lax.broadcasted_iota(jnp.int32, (TB, TB), 0)
        tt = lax.broadcasted_iota(jnp.int32, (TB, TB), 1)
        tri_v[...] = (jj < tt).astype(jnp.float32)

    xb = xb_ref[...]
    wg = wg_ref[...]
    lg = lax.dot_general(wg, xb, (((1,), (1,)), ((), ())),
                         preferred_element_type=jnp.float32)
    m = jnp.max(lg, axis=0, keepdims=True)
    p = jnp.exp(lg - m)
    s = jnp.sum(p, axis=0, keepdims=True)
    w = 1.0 / s

    eio = lax.broadcasted_iota(jnp.int32, (E, 1), 0)
    hit = lg == m
    idx = jnp.min(jnp.where(hit, eio, E), axis=0, keepdims=True)
    ohf = (eio == idx).astype(jnp.float32)
    excl = lax.dot_general(ohf, tri_v[...], (((1,), (0,)), ((), ())),
                           preferred_element_type=jnp.float32)
    carry = cnt_acc[...]
    rank = jnp.sum(ohf * (excl + carry), axis=0,
                   keepdims=True).astype(jnp.int32)
    cnt_acc[...] = carry + excl[:, TB - 1:TB] + ohf[:, TB - 1:TB]
    me_acc[...] = me_acc[...] + jnp.sum(p * w, axis=1, keepdims=True)

    valid = rank < CAP
    slot = idx * CAP_PAD + rank
    dsc_ref[...] = jnp.where(valid, slot, NSLOT)[0]
    dg_ref[...] = jnp.where(valid, slot, CAP)[0]
    w_ref[...] = w[0]

    xpk_ref[...] = _rnd_hi(xb[:, PK:]) | (_rnd_hi(xb[:, :PK]) >> jnp.uint32(16))

    @pl.when(i == n - 1)
    def _():
        me = me_acc[...] / NT
        assign = cnt_acc[...] / NT
        aux_ref[...] = (jnp.sum(me * assign) * (E * 0.01)).reshape(1, 1)


def _route(xf, Wg):
    return pl.pallas_call(
        _route_body,
        grid=(NT // TB,),
        in_specs=[
            pl.BlockSpec((TB, DIM), lambda i: (i, 0)),
            pl.BlockSpec((E, DIM), lambda i: (0, 0)),
        ],
        out_specs=[
            pl.BlockSpec((TB,), lambda i: (i,)),
            pl.BlockSpec((TB,), lambda i: (i,)),
            pl.BlockSpec((TB,), lambda i: (i,)),
            pl.BlockSpec((TB, PK), lambda i: (i, 0)),
            pl.BlockSpec((1, 1), lambda i: (0, 0)),
        ],
        out_shape=[
            jax.ShapeDtypeStruct((NT,), jnp.int32),
            jax.ShapeDtypeStruct((NT,), jnp.int32),
            jax.ShapeDtypeStruct((NT,), jnp.float32),
            jax.ShapeDtypeStruct((NT, PK), jnp.uint32),
            jax.ShapeDtypeStruct((1, 1), jnp.float32),
        ],
        scratch_shapes=[
            pltpu.VMEM((E, 1), jnp.float32),
            pltpu.VMEM((E, 1), jnp.float32),
            pltpu.VMEM((TB, TB), jnp.float32),
        ],
    )(xf, Wg)


_BCH = 2048


def _build_body(dsc_hbm, w_hbm, tok_out, wsl_out, tok_v, wsl_v, idx_st, w_st):
    wid = lax.axis_index("s") * NC + lax.axis_index("c")

    @pl.when(wid == 0)
    def _():
        def init_body(k, c):
            fill = (k * 16 + lax.broadcasted_iota(jnp.int32, (16,), 0)) & (NT - 1)
            tok_v[pl.ds(k * 16, 16)] = fill
            wsl_v[pl.ds(k * 16, 16)] = jnp.zeros((16,), jnp.float32)
            return c

        lax.fori_loop(0, (NSLOT + 16) // 16, init_body, 0)

        for c in range(NT // _BCH):
            pltpu.sync_copy(dsc_hbm.at[pl.ds(c * _BCH, _BCH)], idx_st)
            pltpu.sync_copy(w_hbm.at[pl.ds(c * _BCH, _BCH)], w_st)

            def sc_body(j, carry, c=c):
                idxv = idx_st[pl.ds(j * 16, 16)]
                toks = (c * _BCH + j * 16
                        + lax.broadcasted_iota(jnp.int32, (16,), 0))
                plsc.store_scatter(tok_v, [idxv], toks)
                wv = w_st[pl.ds(j * 16, 16)]
                plsc.store_scatter(wsl_v, [idxv], wv)
                return carry

            lax.fori_loop(0, _BCH // 16, sc_body, 0)

        pltpu.sync_copy(tok_v, tok_out)
        pltpu.sync_copy(wsl_v, wsl_out)


def _build(dst_sc, w):
    return pl.kernel(
        _build_body,
        out_type=[
            jax.ShapeDtypeStruct((NSLOT + 16,), jnp.int32),
            jax.ShapeDtypeStruct((NSLOT + 16,), jnp.float32),
        ],
        mesh=_mesh,
        scratch_types=[
            pltpu.VMEM((NSLOT + 16,), jnp.int32),
            pltpu.VMEM((NSLOT + 16,), jnp.float32),
            pltpu.VMEM((_BCH,), jnp.int32),
            pltpu.VMEM((_BCH,), jnp.float32),
        ],
        compiler_params=_sc_params,
    )(dst_sc, w)


NG = 2
E_G = E // NG
NSLOT_G = NSLOT // NG
_GCH = 216


def _gather_x_body(tok_hbm, xpk_hbm, xe_out, idx_v, rows_v, sem, *, group):
    wid = lax.axis_index("s") * NC + lax.axis_index("c")
    per_w = NSLOT_G // NW

    def body(cc, carry):
        base = wid * per_w + cc * _GCH
        pltpu.sync_copy(tok_hbm.at[pl.ds(group * NSLOT_G + base, _GCH)], idx_v)
        pltpu.async_copy(xpk_hbm.at[idx_v], rows_v, sem).wait()
        pltpu.sync_copy(rows_v, xe_out.at[pl.ds(base, _GCH)])
        return carry

    lax.fori_loop(0, per_w // _GCH, body, 0)


def _gather_x(tok_slot, xpk, group):
    return pl.kernel(
        functools.partial(_gather_x_body, group=group),
        out_type=jax.ShapeDtypeStruct((NSLOT_G, PK), jnp.uint32),
        mesh=_mesh,
        scratch_types=[
            pltpu.VMEM((_GCH,), jnp.int32),
            pltpu.VMEM((_GCH, PK), jnp.uint32),
            pltpu.SemaphoreType.DMA,
        ],
        compiler_params=_sc_params,
    )(tok_slot, xpk)


def _ffn_body(xe_ref, w1_ref, w2_ref, ws_ref, ye_ref):
    xp = xe_ref[0]
    lo = lax.bitcast_convert_type(xp << jnp.uint32(16), jnp.float32)
    hi = lax.bitcast_convert_type(xp & jnp.uint32(0xFFFF0000), jnp.float32)
    xb = jnp.concatenate([lo, hi], axis=1).astype(jnp.bfloat16)
    w1 = w1_ref[0].astype(jnp.bfloat16)
    h = lax.dot_general(xb, w1, (((1,), (1,)), ((), ())),
                        preferred_element_type=jnp.float32)
    h = h * jax.nn.sigmoid(h)
    w2 = w2_ref[0].astype(jnp.bfloat16)
    y = lax.dot_general(h.astype(jnp.bfloat16), w2,
                        (((1,), (1,)), ((), ())),
                        preferred_element_type=jnp.float32)
    ws = ws_ref[0, 0, :].reshape(CAP_PAD, 1)
    ye_ref[0] = y * ws


def _ffn_alias_body(ye_in_ref, xe_ref, w1_ref, w2_ref, ws_ref, ye_ref):
    del ye_in_ref
    _ffn_body(xe_ref, w1_ref, w2_ref, ws_ref, ye_ref)


def _ffn(xe3, W1, W2, wsl3, group, ye_prev):
    eoff = group * E_G
    specs = [
        pl.BlockSpec((1, CAP_PAD, PK), lambda e: (e, 0, 0)),
        pl.BlockSpec((1, HIDDEN, DIM), lambda e, eoff=eoff: (e + eoff, 0, 0)),
        pl.BlockSpec((1, DIM, HIDDEN), lambda e, eoff=eoff: (e + eoff, 0, 0)),
        pl.BlockSpec((1, 1, CAP_PAD), lambda e, eoff=eoff: (e + eoff, 0, 0)),
    ]
    out_spec = pl.BlockSpec(
        (1, CAP_PAD, DIM), lambda e, eoff=eoff: (e + eoff, 0, 0))
    out_shape = jax.ShapeDtypeStruct((E, CAP_PAD, DIM), jnp.float32)
    if group == 0:
        return pl.pallas_call(
            _ffn_body,
            grid=(E_G,),
            in_specs=specs,
            out_specs=out_spec,
            out_shape=out_shape,
        )(xe3, W1, W2, wsl3)
    return pl.pallas_call(
        _ffn_alias_body,
        grid=(E_G,),
        in_specs=[pl.BlockSpec(memory_space=pl.ANY)] + specs,
        out_specs=out_spec,
        out_shape=out_shape,
        input_output_aliases={0: 0},
    )(ye_prev, xe3, W1, W2, wsl3)


_OCH = 64


def _combine_body(dg_hbm, ye_hbm, out_hbm, idx_v, rows_v, sem):
    wid = lax.axis_index("s") * NC + lax.axis_index("c")
    per_w = NT // NW

    def body(cc, carry):
        base = wid * per_w + cc * _OCH
        pltpu.sync_copy(dg_hbm.at[pl.ds(base, _OCH)], idx_v)
        pltpu.async_copy(ye_hbm.at[idx_v], rows_v, sem).wait()
        pltpu.sync_copy(rows_v, out_hbm.at[pl.ds(base, _OCH)])
        return carry

    lax.fori_loop(0, per_w // _OCH, body, 0)


def _combine(dst_g, ye):
    return pl.kernel(
        _combine_body,
        out_type=jax.ShapeDtypeStruct((NT, DIM), jnp.float32),
        mesh=_mesh,
        scratch_types=[
            pltpu.VMEM((_OCH,), jnp.int32),
            pltpu.VMEM((_OCH, DIM), jnp.float32),
            pltpu.SemaphoreType.DMA,
        ],
        compiler_params=_sc_params,
    )(dst_g, ye)


def kernel(x, Wg, W1, W2):
    Bx, Tx, D = x.shape
    xf = x.reshape(Bx * Tx, D)
    dst_sc, dst_g, w, xpk, aux = _route(xf, Wg)
    tok_slot, w_slot = _build(dst_sc, w)
    wsl3 = w_slot[:NSLOT].reshape(E, 1, CAP_PAD)
    ye = None
    for g in range(NG):
        xe_g = _gather_x(tok_slot, xpk, g)
        ye = _ffn(xe_g.reshape(E_G, CAP_PAD, PK), W1, W2, wsl3, g, ye)
    out = _combine(dst_g, ye.reshape(NSLOT, D))
    return out.reshape(Bx, Tx, D), aux[0, 0]


# speedup vs baseline: 13.8534x; 1.0514x over previous
import functools

import jax
import jax.numpy as jnp
from jax import lax
from jax.experimental import pallas as pl
from jax.experimental.pallas import tpu as pltpu
from jax.experimental.pallas import tpu_sc as plsc

B, T, DIM = 4, 8192, 768
E = 64
HIDDEN = 2048
NT = B * T
CAP = int(1.25 * NT / E) + 1
CAP_PAD = (CAP // 8 + 1) * 8
NSLOT = E * CAP_PAD
TB = 512

NC, NS = 2, 16
NW = NC * NS

_mesh = plsc.VectorSubcoreMesh(core_axis_name="c", subcore_axis_name="s")
_sc_params = pltpu.CompilerParams(needs_layout_passes=False)


PK = DIM // 2


def _rnd_hi(v):
    u = lax.bitcast_convert_type(v, jnp.uint32)
    u = u + jnp.uint32(0x7FFF) + ((u >> jnp.uint32(16)) & jnp.uint32(1))
    return u & jnp.uint32(0xFFFF0000)


def _route_body(xb_ref, wg_ref, dsc_ref, dg_ref, w_ref, xpk_ref, aux_ref,
                me_acc, cnt_acc, tri_v):
    i = pl.program_id(0)
    n = pl.num_programs(0)

    @pl.when(i == 0)
    def _():
        me_acc[...] = jnp.zeros_like(me_acc)
        cnt_acc[...] = jnp.zeros_like(cnt_acc)
        jj = lax.broadcasted_iota(jnp.int32, (TB, TB), 0)
        tt = lax.broadcasted_iota(jnp.int32, (TB, TB), 1)
        tri_v[...] = (jj < tt).astype(jnp.float32)

    xb = xb_ref[...]
    wg = wg_ref[...]
    lg = lax.dot_general(wg, xb, (((1,), (1,)), ((), ())),
                         preferred_element_type=jnp.float32)
    m = jnp.max(lg, axis=0, keepdims=True)
    p = jnp.exp(lg - m)
    s = jnp.sum(p, axis=0, keepdims=True)
    w = 1.0 / s

    eio = lax.broadcasted_iota(jnp.int32, (E, 1), 0)
    hit = lg == m
    idx = jnp.min(jnp.where(hit, eio, E), axis=0, keepdims=True)
    ohf = (eio == idx).astype(jnp.float32)
    excl = lax.dot_general(ohf, tri_v[...], (((1,), (0,)), ((), ())),
                           preferred_element_type=jnp.float32)
    carry = cnt_acc[...]
    rank = jnp.sum(ohf * (excl + carry), axis=0,
                   keepdims=True).astype(jnp.int32)
    cnt_acc[...] = carry + excl[:, TB - 1:TB] + ohf[:, TB - 1:TB]
    me_acc[...] = me_acc[...] + jnp.sum(p * w, axis=1, keepdims=True)

    valid = rank < CAP
    slot = idx * CAP_PAD + rank
    dsc_ref[...] = jnp.where(valid, slot, NSLOT)[0]
    dg_ref[...] = jnp.where(valid, slot, CAP)[0]
    w_ref[...] = w[0]

    xpk_ref[...] = _rnd_hi(xb[:, PK:]) | (_rnd_hi(xb[:, :PK]) >> jnp.uint32(16))

    @pl.when(i == n - 1)
    def _():
        me = me_acc[...] / NT
        assign = cnt_acc[...] / NT
        aux_ref[...] = (jnp.sum(me * assign) * (E * 0.01)).reshape(1, 1)


def _route(xf, Wg):
    return pl.pallas_call(
        _route_body,
        grid=(NT // TB,),
        in_specs=[
            pl.BlockSpec((TB, DIM), lambda i: (i, 0)),
            pl.BlockSpec((E, DIM), lambda i: (0, 0)),
        ],
        out_specs=[
            pl.BlockSpec((TB,), lambda i: (i,)),
            pl.BlockSpec((TB,), lambda i: (i,)),
            pl.BlockSpec((TB,), lambda i: (i,)),
            pl.BlockSpec((TB, PK), lambda i: (i, 0)),
            pl.BlockSpec((1, 1), lambda i: (0, 0)),
        ],
        out_shape=[
            jax.ShapeDtypeStruct((NT,), jnp.int32),
            jax.ShapeDtypeStruct((NT,), jnp.int32),
            jax.ShapeDtypeStruct((NT,), jnp.float32),
            jax.ShapeDtypeStruct((NT, PK), jnp.uint32),
            jax.ShapeDtypeStruct((1, 1), jnp.float32),
        ],
        scratch_shapes=[
            pltpu.VMEM((E, 1), jnp.float32),
            pltpu.VMEM((E, 1), jnp.float32),
            pltpu.VMEM((TB, TB), jnp.float32),
        ],
    )(xf, Wg)


RNG_T = 1304
NSLOT_PAD = RNG_T * NW


def _build_body(dsc_hbm, w_hbm, tok_out, wsl_out, tok_v, wsl_v, idx_st, w_st):
    wid = lax.axis_index("s") * NC + lax.axis_index("c")
    base = wid * RNG_T

    def init_body(k, c):
        fill = (base + k * 16
                + lax.broadcasted_iota(jnp.int32, (16,), 0)) & (NT - 1)
        tok_v[pl.ds(k * 16, 16)] = fill
        wsl_v[pl.ds(k * 16, 16)] = jnp.zeros((16,), jnp.float32)
        return c

    lax.fori_loop(0, RNG_T // 16, init_body, 0)
    pltpu.sync_copy(dsc_hbm, idx_st)
    pltpu.sync_copy(w_hbm, w_st)

    def sc_body(j, carry):
        idxv = idx_st[pl.ds(j * 16, 16)]
        local = idxv - base
        keep = (local >= 0) & (local < RNG_T)
        lcl = jnp.minimum(jnp.maximum(local, 0), RNG_T - 1)
        toks = j * 16 + lax.broadcasted_iota(jnp.int32, (16,), 0)
        plsc.store_scatter(tok_v, [lcl], toks, mask=keep)
        wv = w_st[pl.ds(j * 16, 16)]
        plsc.store_scatter(wsl_v, [lcl], wv, mask=keep)
        return carry

    lax.fori_loop(0, NT // 16, sc_body, 0)
    pltpu.sync_copy(tok_v, tok_out.at[pl.ds(base, RNG_T)])
    pltpu.sync_copy(wsl_v, wsl_out.at[pl.ds(base, RNG_T)])


def _build(dst_sc, w):
    return pl.kernel(
        _build_body,
        out_type=[
            jax.ShapeDtypeStruct((NSLOT_PAD,), jnp.int32),
            jax.ShapeDtypeStruct((NSLOT_PAD,), jnp.float32),
        ],
        mesh=_mesh,
        scratch_types=[
            pltpu.VMEM((RNG_T,), jnp.int32),
            pltpu.VMEM((RNG_T,), jnp.float32),
            pltpu.VMEM((NT,), jnp.int32),
            pltpu.VMEM((NT,), jnp.float32),
        ],
        compiler_params=_sc_params,
    )(dst_sc, w)


NG = 2
E_G = E // NG
NSLOT_G = NSLOT // NG
_GCH = 216


def _gather_x_body(tok_hbm, xpk_hbm, xe_out, idx_v, rows_v, sem, *, group):
    wid = lax.axis_index("s") * NC + lax.axis_index("c")
    per_w = NSLOT_G // NW

    def body(cc, carry):
        base = wid * per_w + cc * _GCH
        pltpu.sync_copy(tok_hbm.at[pl.ds(group * NSLOT_G + base, _GCH)], idx_v)
        pltpu.async_copy(xpk_hbm.at[idx_v], rows_v, sem).wait()
        pltpu.sync_copy(rows_v, xe_out.at[pl.ds(base, _GCH)])
        return carry

    lax.fori_loop(0, per_w // _GCH, body, 0)


def _gather_x(tok_slot, xpk, group):
    return pl.kernel(
        functools.partial(_gather_x_body, group=group),
        out_type=jax.ShapeDtypeStruct((NSLOT_G, PK), jnp.uint32),
        mesh=_mesh,
        scratch_types=[
            pltpu.VMEM((_GCH,), jnp.int32),
            pltpu.VMEM((_GCH, PK), jnp.uint32),
            pltpu.SemaphoreType.DMA,
        ],
        compiler_params=_sc_params,
    )(tok_slot, xpk)


def _ffn_body(xe_ref, w1_ref, w2_ref, ws_ref, ye_ref):
    xp = xe_ref[0]
    lo = lax.bitcast_convert_type(xp << jnp.uint32(16), jnp.float32)
    hi = lax.bitcast_convert_type(xp & jnp.uint32(0xFFFF0000), jnp.float32)
    xb = jnp.concatenate([lo, hi], axis=1).astype(jnp.bfloat16)
    w1 = w1_ref[0].astype(jnp.bfloat16)
    h = lax.dot_general(xb, w1, (((1,), (1,)), ((), ())),
                        preferred_element_type=jnp.float32)
    h = h * jax.nn.sigmoid(h)
    w2 = w2_ref[0].astype(jnp.bfloat16)
    y = lax.dot_general(h.astype(jnp.bfloat16), w2,
                        (((1,), (1,)), ((), ())),
                        preferred_element_type=jnp.float32)
    ws = ws_ref[0, 0, :].reshape(CAP_PAD, 1)
    ye_ref[0] = y * ws


def _ffn_alias_body(ye_in_ref, xe_ref, w1_ref, w2_ref, ws_ref, ye_ref):
    del ye_in_ref
    _ffn_body(xe_ref, w1_ref, w2_ref, ws_ref, ye_ref)


def _ffn(xe3, W1, W2, wsl3, group, ye_prev):
    eoff = group * E_G
    specs = [
        pl.BlockSpec((1, CAP_PAD, PK), lambda e: (e, 0, 0)),
        pl.BlockSpec((1, HIDDEN, DIM), lambda e, eoff=eoff: (e + eoff, 0, 0)),
        pl.BlockSpec((1, DIM, HIDDEN), lambda e, eoff=eoff: (e + eoff, 0, 0)),
        pl.BlockSpec((1, 1, CAP_PAD), lambda e, eoff=eoff: (e + eoff, 0, 0)),
    ]
    out_spec = pl.BlockSpec(
        (1, CAP_PAD, DIM), lambda e, eoff=eoff: (e + eoff, 0, 0))
    out_shape = jax.ShapeDtypeStruct((E, CAP_PAD, DIM), jnp.float32)
    if group == 0:
        return pl.pallas_call(
            _ffn_body,
            grid=(E_G,),
            in_specs=specs,
            out_specs=out_spec,
            out_shape=out_shape,
        )(xe3, W1, W2, wsl3)
    return pl.pallas_call(
        _ffn_alias_body,
        grid=(E_G,),
        in_specs=[pl.BlockSpec(memory_space=pl.ANY)] + specs,
        out_specs=out_spec,
        out_shape=out_shape,
        input_output_aliases={0: 0},
    )(ye_prev, xe3, W1, W2, wsl3)


_OCH = 128


def _combine_body(dg_hbm, ye_hbm, out_hbm, idx_v, rows_v, sem):
    wid = lax.axis_index("s") * NC + lax.axis_index("c")
    per_w = NT // NW

    def body(cc, carry):
        base = wid * per_w + cc * _OCH
        pltpu.sync_copy(dg_hbm.at[pl.ds(base, _OCH)], idx_v)
        pltpu.async_copy(ye_hbm.at[idx_v], rows_v, sem).wait()
        pltpu.sync_copy(rows_v, out_hbm.at[pl.ds(base, _OCH)])
        return carry

    lax.fori_loop(0, per_w // _OCH, body, 0)


def _combine(dst_g, ye):
    return pl.kernel(
        _combine_body,
        out_type=jax.ShapeDtypeStruct((NT, DIM), jnp.float32),
        mesh=_mesh,
        scratch_types=[
            pltpu.VMEM((_OCH,), jnp.int32),
            pltpu.VMEM((_OCH, DIM), jnp.float32),
            pltpu.SemaphoreType.DMA,
        ],
        compiler_params=_sc_params,
    )(dst_g, ye)


def kernel(x, Wg, W1, W2):
    Bx, Tx, D = x.shape
    xf = x.reshape(Bx * Tx, D)
    dst_sc, dst_g, w, xpk, aux = _route(xf, Wg)
    tok_slot, w_slot = _build(dst_sc, w)
    wsl3 = w_slot[:NSLOT].reshape(E, 1, CAP_PAD)
    ye = None
    for g in range(NG):
        xe_g = _gather_x(tok_slot, xpk, g)
        ye = _ffn(xe_g.reshape(E_G, CAP_PAD, PK), W1, W2, wsl3, g, ye)
    out = _combine(dst_g, ye.reshape(NSLOT, D))
    return out.reshape(Bx, Tx, D), aux[0, 0]


# speedup vs baseline: 14.2908x; 1.0316x over previous
import functools

import jax
import jax.numpy as jnp
from jax import lax
from jax.experimental import pallas as pl
from jax.experimental.pallas import tpu as pltpu
from jax.experimental.pallas import tpu_sc as plsc

B, T, DIM = 4, 8192, 768
E = 64
HIDDEN = 2048
NT = B * T
CAP = int(1.25 * NT / E) + 1
CAP_PAD = (CAP // 8 + 1) * 8
NSLOT = E * CAP_PAD
TB = 1024

NC, NS = 2, 16
NW = NC * NS

_mesh = plsc.VectorSubcoreMesh(core_axis_name="c", subcore_axis_name="s")
_sc_params = pltpu.CompilerParams(needs_layout_passes=False)


PK = DIM // 2


def _rnd_hi(v):
    u = lax.bitcast_convert_type(v, jnp.uint32)
    u = u + jnp.uint32(0x7FFF) + ((u >> jnp.uint32(16)) & jnp.uint32(1))
    return u & jnp.uint32(0xFFFF0000)


def _route_body(xb_ref, wg_ref, dsc_ref, dg_ref, w_ref, xpk_ref, aux_ref,
                me_acc, cnt_acc, tri_v):
    i = pl.program_id(0)
    n = pl.num_programs(0)

    @pl.when(i == 0)
    def _():
        me_acc[...] = jnp.zeros_like(me_acc)
        cnt_acc[...] = jnp.zeros_like(cnt_acc)
        jj = lax.broadcasted_iota(jnp.int32, (TB, TB), 0)
        tt = lax.broadcasted_iota(jnp.int32, (TB, TB), 1)
        tri_v[...] = (jj < tt).astype(jnp.float32)

    xb = xb_ref[...]
    wg = wg_ref[...]
    lg = lax.dot_general(wg, xb, (((1,), (1,)), ((), ())),
                         preferred_element_type=jnp.float32)
    m = jnp.max(lg, axis=0, keepdims=True)
    p = jnp.exp(lg - m)
    s = jnp.sum(p, axis=0, keepdims=True)
    w = 1.0 / s

    eio = lax.broadcasted_iota(jnp.int32, (E, 1), 0)
    hit = lg == m
    idx = jnp.min(jnp.where(hit, eio, E), axis=0, keepdims=True)
    ohf = (eio == idx).astype(jnp.float32)
    excl = lax.dot_general(ohf, tri_v[...], (((1,), (0,)), ((), ())),
                           preferred_element_type=jnp.float32)
    carry = cnt_acc[...]
    rank = jnp.sum(ohf * (excl + carry), axis=0,
                   keepdims=True).astype(jnp.int32)
    cnt_acc[...] = carry + excl[:, TB - 1:TB] + ohf[:, TB - 1:TB]
    me_acc[...] = me_acc[...] + jnp.sum(p * w, axis=1, keepdims=True)

    valid = rank < CAP
    slot = idx * CAP_PAD + rank
    dsc_ref[...] = jnp.where(valid, slot, NSLOT)[0]
    dg_ref[...] = jnp.where(valid, slot, CAP)[0]
    w_ref[...] = w[0]

    xpk_ref[...] = _rnd_hi(xb[:, PK:]) | (_rnd_hi(xb[:, :PK]) >> jnp.uint32(16))

    @pl.when(i == n - 1)
    def _():
        me = me_acc[...] / NT
        assign = cnt_acc[...] / NT
        aux_ref[...] = (jnp.sum(me * assign) * (E * 0.01)).reshape(1, 1)


def _route(xf, Wg):
    return pl.pallas_call(
        _route_body,
        grid=(NT // TB,),
        in_specs=[
            pl.BlockSpec((TB, DIM), lambda i: (i, 0)),
            pl.BlockSpec((E, DIM), lambda i: (0, 0)),
        ],
        out_specs=[
            pl.BlockSpec((TB,), lambda i: (i,)),
            pl.BlockSpec((TB,), lambda i: (i,)),
            pl.BlockSpec((TB,), lambda i: (i,)),
            pl.BlockSpec((TB, PK), lambda i: (i, 0)),
            pl.BlockSpec((1, 1), lambda i: (0, 0)),
        ],
        out_shape=[
            jax.ShapeDtypeStruct((NT,), jnp.int32),
            jax.ShapeDtypeStruct((NT,), jnp.int32),
            jax.ShapeDtypeStruct((NT,), jnp.float32),
            jax.ShapeDtypeStruct((NT, PK), jnp.uint32),
            jax.ShapeDtypeStruct((1, 1), jnp.float32),
        ],
        scratch_shapes=[
            pltpu.VMEM((E, 1), jnp.float32),
            pltpu.VMEM((E, 1), jnp.float32),
            pltpu.VMEM((TB, TB), jnp.float32),
        ],
    )(xf, Wg)


RNG_T = 1304
NSLOT_PAD = RNG_T * NW


def _build_body(dsc_hbm, w_hbm, tok_out, wsl_out, tok_v, wsl_v, idx_st, w_st):
    wid = lax.axis_index("s") * NC + lax.axis_index("c")
    base = wid * RNG_T

    def init_body(k, c):
        fill = (base + k * 16
                + lax.broadcasted_iota(jnp.int32, (16,), 0)) & (NT - 1)
        tok_v[pl.ds(k * 16, 16)] = fill
        wsl_v[pl.ds(k * 16, 16)] = jnp.zeros((16,), jnp.float32)
        return c

    lax.fori_loop(0, RNG_T // 16, init_body, 0)
    pltpu.sync_copy(dsc_hbm, idx_st)
    pltpu.sync_copy(w_hbm, w_st)

    def sc_body(j, carry):
        idxv = idx_st[pl.ds(j * 16, 16)]
        local = idxv - base
        keep = (local >= 0) & (local < RNG_T)
        lcl = jnp.minimum(jnp.maximum(local, 0), RNG_T - 1)
        toks = j * 16 + lax.broadcasted_iota(jnp.int32, (16,), 0)
        plsc.store_scatter(tok_v, [lcl], toks, mask=keep)
        wv = w_st[pl.ds(j * 16, 16)]
        plsc.store_scatter(wsl_v, [lcl], wv, mask=keep)
        return carry

    lax.fori_loop(0, NT // 16, sc_body, 0)
    pltpu.sync_copy(tok_v, tok_out.at[pl.ds(base, RNG_T)])
    pltpu.sync_copy(wsl_v, wsl_out.at[pl.ds(base, RNG_T)])


def _build(dst_sc, w):
    return pl.kernel(
        _build_body,
        out_type=[
            jax.ShapeDtypeStruct((NSLOT_PAD,), jnp.int32),
            jax.ShapeDtypeStruct((NSLOT_PAD,), jnp.float32),
        ],
        mesh=_mesh,
        scratch_types=[
            pltpu.VMEM((RNG_T,), jnp.int32),
            pltpu.VMEM((RNG_T,), jnp.float32),
            pltpu.VMEM((NT,), jnp.int32),
            pltpu.VMEM((NT,), jnp.float32),
        ],
        compiler_params=_sc_params,
    )(dst_sc, w)


NG = 2
E_G = E // NG
NSLOT_G = NSLOT // NG
_GCH = 216


def _gather_x_body(tok_hbm, xpk_hbm, xe_out, idx_v, rows_v, sem, *, group):
    wid = lax.axis_index("s") * NC + lax.axis_index("c")
    per_w = NSLOT_G // NW

    def body(cc, carry):
        base = wid * per_w + cc * _GCH
        pltpu.sync_copy(tok_hbm.at[pl.ds(group * NSLOT_G + base, _GCH)], idx_v)
        pltpu.async_copy(xpk_hbm.at[idx_v], rows_v, sem).wait()
        pltpu.sync_copy(rows_v, xe_out.at[pl.ds(base, _GCH)])
        return carry

    lax.fori_loop(0, per_w // _GCH, body, 0)


def _gather_x(tok_slot, xpk, group):
    return pl.kernel(
        functools.partial(_gather_x_body, group=group),
        out_type=jax.ShapeDtypeStruct((NSLOT_G, PK), jnp.uint32),
        mesh=_mesh,
        scratch_types=[
            pltpu.VMEM((_GCH,), jnp.int32),
            pltpu.VMEM((_GCH, PK), jnp.uint32),
            pltpu.SemaphoreType.DMA,
        ],
        compiler_params=_sc_params,
    )(tok_slot, xpk)


def _ffn_body(xe_ref, w1_ref, w2_ref, ws_ref, ye_ref):
    xp = xe_ref[0]
    lo = lax.bitcast_convert_type(xp << jnp.uint32(16), jnp.float32)
    hi = lax.bitcast_convert_type(xp & jnp.uint32(0xFFFF0000), jnp.float32)
    xb = jnp.concatenate([lo, hi], axis=1).astype(jnp.bfloat16)
    w1 = w1_ref[0].astype(jnp.bfloat16)
    h = lax.dot_general(xb, w1, (((1,), (1,)), ((), ())),
                        preferred_element_type=jnp.float32)
    h = h * jax.nn.sigmoid(h)
    w2 = w2_ref[0].astype(jnp.bfloat16)
    y = lax.dot_general(h.astype(jnp.bfloat16), w2,
                        (((1,), (1,)), ((), ())),
                        preferred_element_type=jnp.float32)
    ws = ws_ref[0, 0, :].reshape(CAP_PAD, 1)
    ye_ref[0] = y * ws


def _ffn_alias_body(ye_in_ref, xe_ref, w1_ref, w2_ref, ws_ref, ye_ref):
    del ye_in_ref
    _ffn_body(xe_ref, w1_ref, w2_ref, ws_ref, ye_ref)


def _ffn(xe3, W1, W2, wsl3, group, ye_prev):
    eoff = group * E_G
    specs = [
        pl.BlockSpec((1, CAP_PAD, PK), lambda e: (e, 0, 0)),
        pl.BlockSpec((1, HIDDEN, DIM), lambda e, eoff=eoff: (e + eoff, 0, 0)),
        pl.BlockSpec((1, DIM, HIDDEN), lambda e, eoff=eoff: (e + eoff, 0, 0)),
        pl.BlockSpec((1, 1, CAP_PAD), lambda e, eoff=eoff: (e + eoff, 0, 0)),
    ]
    out_spec = pl.BlockSpec(
        (1, CAP_PAD, DIM), lambda e, eoff=eoff: (e + eoff, 0, 0))
    out_shape = jax.ShapeDtypeStruct((E, CAP_PAD, DIM), jnp.float32)
    if group == 0:
        return pl.pallas_call(
            _ffn_body,
            grid=(E_G,),
            in_specs=specs,
            out_specs=out_spec,
            out_shape=out_shape,
        )(xe3, W1, W2, wsl3)
    return pl.pallas_call(
        _ffn_alias_body,
        grid=(E_G,),
        in_specs=[pl.BlockSpec(memory_space=pl.ANY)] + specs,
        out_specs=out_spec,
        out_shape=out_shape,
        input_output_aliases={0: 0},
    )(ye_prev, xe3, W1, W2, wsl3)


_OCH = 128


def _combine_body(dg_hbm, ye_hbm, out_hbm, idx_v, rows_v, sem):
    wid = lax.axis_index("s") * NC + lax.axis_index("c")
    per_w = NT // NW

    def body(cc, carry):
        base = wid * per_w + cc * _OCH
        pltpu.sync_copy(dg_hbm.at[pl.ds(base, _OCH)], idx_v)
        pltpu.async_copy(ye_hbm.at[idx_v], rows_v, sem).wait()
        pltpu.sync_copy(rows_v, out_hbm.at[pl.ds(base, _OCH)])
        return carry

    lax.fori_loop(0, per_w // _OCH, body, 0)


def _combine(dst_g, ye):
    return pl.kernel(
        _combine_body,
        out_type=jax.ShapeDtypeStruct((NT, DIM), jnp.float32),
        mesh=_mesh,
        scratch_types=[
            pltpu.VMEM((_OCH,), jnp.int32),
            pltpu.VMEM((_OCH, DIM), jnp.float32),
            pltpu.SemaphoreType.DMA,
        ],
        compiler_params=_sc_params,
    )(dst_g, ye)


def kernel(x, Wg, W1, W2):
    Bx, Tx, D = x.shape
    xf = x.reshape(Bx * Tx, D)
    dst_sc, dst_g, w, xpk, aux = _route(xf, Wg)
    tok_slot, w_slot = _build(dst_sc, w)
    wsl3 = w_slot[:NSLOT].reshape(E, 1, CAP_PAD)
    ye = None
    for g in range(NG):
        xe_g = _gather_x(tok_slot, xpk, g)
        ye = _ffn(xe_g.reshape(E_G, CAP_PAD, PK), W1, W2, wsl3, g, ye)
    out = _combine(dst_g, ye.reshape(NSLOT, D))
    return out.reshape(Bx, Tx, D), aux[0, 0]


# speedup vs baseline: 14.3878x; 1.0068x over previous
import functools

import jax
import jax.numpy as jnp
from jax import lax
from jax.experimental import pallas as pl
from jax.experimental.pallas import tpu as pltpu
from jax.experimental.pallas import tpu_sc as plsc

B, T, DIM = 4, 8192, 768
E = 64
HIDDEN = 2048
NT = B * T
CAP = int(1.25 * NT / E) + 1
CAP_PAD = (CAP // 8 + 1) * 8
NSLOT = E * CAP_PAD
TB = 2048

NC, NS = 2, 16
NW = NC * NS

_mesh = plsc.VectorSubcoreMesh(core_axis_name="c", subcore_axis_name="s")
_sc_params = pltpu.CompilerParams(needs_layout_passes=False)


PK = DIM // 2


def _rnd_hi(v):
    u = lax.bitcast_convert_type(v, jnp.uint32)
    u = u + jnp.uint32(0x7FFF) + ((u >> jnp.uint32(16)) & jnp.uint32(1))
    return u & jnp.uint32(0xFFFF0000)


def _route_body(xb_ref, wg_ref, dsc_ref, dg_ref, w_ref, xpk_ref, aux_ref,
                me_acc, cnt_acc, tri_v):
    i = pl.program_id(0)
    n = pl.num_programs(0)

    @pl.when(i == 0)
    def _():
        me_acc[...] = jnp.zeros_like(me_acc)
        cnt_acc[...] = jnp.zeros_like(cnt_acc)
        jj = lax.broadcasted_iota(jnp.int32, (TB, TB), 0)
        tt = lax.broadcasted_iota(jnp.int32, (TB, TB), 1)
        tri_v[...] = (jj < tt).astype(jnp.float32)

    xb = xb_ref[...]
    wg = wg_ref[...]
    lg = lax.dot_general(wg, xb, (((1,), (1,)), ((), ())),
                         preferred_element_type=jnp.float32)
    m = jnp.max(lg, axis=0, keepdims=True)
    p = jnp.exp(lg - m)
    s = jnp.sum(p, axis=0, keepdims=True)
    w = 1.0 / s

    eio = lax.broadcasted_iota(jnp.int32, (E, 1), 0)
    hit = lg == m
    idx = jnp.min(jnp.where(hit, eio, E), axis=0, keepdims=True)
    ohf = (eio == idx).astype(jnp.float32)
    excl = lax.dot_general(ohf, tri_v[...], (((1,), (0,)), ((), ())),
                           preferred_element_type=jnp.float32)
    carry = cnt_acc[...]
    rank = jnp.sum(ohf * (excl + carry), axis=0,
                   keepdims=True).astype(jnp.int32)
    cnt_acc[...] = carry + excl[:, TB - 1:TB] + ohf[:, TB - 1:TB]
    me_acc[...] = me_acc[...] + jnp.sum(p * w, axis=1, keepdims=True)

    valid = rank < CAP
    slot = idx * CAP_PAD + rank
    dsc_ref[...] = jnp.where(valid, slot, NSLOT)[0]
    dg_ref[...] = jnp.where(valid, slot, CAP)[0]
    w_ref[...] = w[0]

    xpk_ref[...] = _rnd_hi(xb[:, PK:]) | (_rnd_hi(xb[:, :PK]) >> jnp.uint32(16))

    @pl.when(i == n - 1)
    def _():
        me = me_acc[...] / NT
        assign = cnt_acc[...] / NT
        aux_ref[...] = (jnp.sum(me * assign) * (E * 0.01)).reshape(1, 1)


def _route(xf, Wg):
    return pl.pallas_call(
        _route_body,
        grid=(NT // TB,),
        in_specs=[
            pl.BlockSpec((TB, DIM), lambda i: (i, 0)),
            pl.BlockSpec((E, DIM), lambda i: (0, 0)),
        ],
        out_specs=[
            pl.BlockSpec((TB,), lambda i: (i,)),
            pl.BlockSpec((TB,), lambda i: (i,)),
            pl.BlockSpec((TB,), lambda i: (i,)),
            pl.BlockSpec((TB, PK), lambda i: (i, 0)),
            pl.BlockSpec((1, 1), lambda i: (0, 0)),
        ],
        out_shape=[
            jax.ShapeDtypeStruct((NT,), jnp.int32),
            jax.ShapeDtypeStruct((NT,), jnp.int32),
            jax.ShapeDtypeStruct((NT,), jnp.float32),
            jax.ShapeDtypeStruct((NT, PK), jnp.uint32),
            jax.ShapeDtypeStruct((1, 1), jnp.float32),
        ],
        scratch_shapes=[
            pltpu.VMEM((E, 1), jnp.float32),
            pltpu.VMEM((E, 1), jnp.float32),
            pltpu.VMEM((TB, TB), jnp.float32),
        ],
    )(xf, Wg)


RNG_T = 1304
NSLOT_PAD = RNG_T * NW


def _build_body(dsc_hbm, w_hbm, tok_out, wsl_out, tok_v, wsl_v, idx_st, w_st):
    wid = lax.axis_index("s") * NC + lax.axis_index("c")
    base = wid * RNG_T

    def init_body(k, c):
        fill = (base + k * 16
                + lax.broadcasted_iota(jnp.int32, (16,), 0)) & (NT - 1)
        tok_v[pl.ds(k * 16, 16)] = fill
        wsl_v[pl.ds(k * 16, 16)] = jnp.zeros((16,), jnp.float32)
        return c

    lax.fori_loop(0, RNG_T // 16, init_body, 0)
    pltpu.sync_copy(dsc_hbm, idx_st)
    pltpu.sync_copy(w_hbm, w_st)

    def sc_body(j, carry):
        idxv = idx_st[pl.ds(j * 16, 16)]
        local = idxv - base
        keep = (local >= 0) & (local < RNG_T)
        lcl = jnp.minimum(jnp.maximum(local, 0), RNG_T - 1)
        toks = j * 16 + lax.broadcasted_iota(jnp.int32, (16,), 0)
        plsc.store_scatter(tok_v, [lcl], toks, mask=keep)
        wv = w_st[pl.ds(j * 16, 16)]
        plsc.store_scatter(wsl_v, [lcl], wv, mask=keep)
        return carry

    lax.fori_loop(0, NT // 16, sc_body, 0)
    pltpu.sync_copy(tok_v, tok_out.at[pl.ds(base, RNG_T)])
    pltpu.sync_copy(wsl_v, wsl_out.at[pl.ds(base, RNG_T)])


def _build(dst_sc, w):
    return pl.kernel(
        _build_body,
        out_type=[
            jax.ShapeDtypeStruct((NSLOT_PAD,), jnp.int32),
            jax.ShapeDtypeStruct((NSLOT_PAD,), jnp.float32),
        ],
        mesh=_mesh,
        scratch_types=[
            pltpu.VMEM((RNG_T,), jnp.int32),
            pltpu.VMEM((RNG_T,), jnp.float32),
            pltpu.VMEM((NT,), jnp.int32),
            pltpu.VMEM((NT,), jnp.float32),
        ],
        compiler_params=_sc_params,
    )(dst_sc, w)


NG = 2
E_G = E // NG
NSLOT_G = NSLOT // NG
_GCH = 216


def _gather_x_body(tok_hbm, xpk_hbm, xe_out, idx_v, rows_v, sem, *, group):
    wid = lax.axis_index("s") * NC + lax.axis_index("c")
    per_w = NSLOT_G // NW

    def body(cc, carry):
        base = wid * per_w + cc * _GCH
        pltpu.sync_copy(tok_hbm.at[pl.ds(group * NSLOT_G + base, _GCH)], idx_v)
        pltpu.async_copy(xpk_hbm.at[idx_v], rows_v, sem).wait()
        pltpu.sync_copy(rows_v, xe_out.at[pl.ds(base, _GCH)])
        return carry

    lax.fori_loop(0, per_w // _GCH, body, 0)


def _gather_x(tok_slot, xpk, group):
    return pl.kernel(
        functools.partial(_gather_x_body, group=group),
        out_type=jax.ShapeDtypeStruct((NSLOT_G, PK), jnp.uint32),
        mesh=_mesh,
        scratch_types=[
            pltpu.VMEM((_GCH,), jnp.int32),
            pltpu.VMEM((_GCH, PK), jnp.uint32),
            pltpu.SemaphoreType.DMA,
        ],
        compiler_params=_sc_params,
    )(tok_slot, xpk)


def _ffn_body(xe_ref, w1_ref, w2_ref, ws_ref, ye_ref):
    xp = xe_ref[0]
    lo = lax.bitcast_convert_type(xp << jnp.uint32(16), jnp.float32)
    hi = lax.bitcast_convert_type(xp & jnp.uint32(0xFFFF0000), jnp.float32)
    xb = jnp.concatenate([lo, hi], axis=1).astype(jnp.bfloat16)
    w1 = w1_ref[0].astype(jnp.bfloat16)
    h = lax.dot_general(xb, w1, (((1,), (1,)), ((), ())),
                        preferred_element_type=jnp.float32)
    h = h * jax.nn.sigmoid(h)
    w2 = w2_ref[0].astype(jnp.bfloat16)
    y = lax.dot_general(h.astype(jnp.bfloat16), w2,
                        (((1,), (1,)), ((), ())),
                        preferred_element_type=jnp.float32)
    ws = ws_ref[0, 0, :].reshape(CAP_PAD, 1)
    ye_ref[0] = y * ws


def _ffn_alias_body(ye_in_ref, xe_ref, w1_ref, w2_ref, ws_ref, ye_ref):
    del ye_in_ref
    _ffn_body(xe_ref, w1_ref, w2_ref, ws_ref, ye_ref)


def _ffn(xe3, W1, W2, wsl3, group, ye_prev):
    eoff = group * E_G
    specs = [
        pl.BlockSpec((1, CAP_PAD, PK), lambda e: (e, 0, 0)),
        pl.BlockSpec((1, HIDDEN, DIM), lambda e, eoff=eoff: (e + eoff, 0, 0)),
        pl.BlockSpec((1, DIM, HIDDEN), lambda e, eoff=eoff: (e + eoff, 0, 0)),
        pl.BlockSpec((1, 1, CAP_PAD), lambda e, eoff=eoff: (e + eoff, 0, 0)),
    ]
    out_spec = pl.BlockSpec(
        (1, CAP_PAD, DIM), lambda e, eoff=eoff: (e + eoff, 0, 0))
    out_shape = jax.ShapeDtypeStruct((E, CAP_PAD, DIM), jnp.float32)
    if group == 0:
        return pl.pallas_call(
            _ffn_body,
            grid=(E_G,),
            in_specs=specs,
            out_specs=out_spec,
            out_shape=out_shape,
        )(xe3, W1, W2, wsl3)
    return pl.pallas_call(
        _ffn_alias_body,
        grid=(E_G,),
        in_specs=[pl.BlockSpec(memory_space=pl.ANY)] + specs,
        out_specs=out_spec,
        out_shape=out_shape,
        input_output_aliases={0: 0},
    )(ye_prev, xe3, W1, W2, wsl3)


_OCH = 128


def _combine_body(dg_hbm, ye_hbm, out_hbm, idx_v, rows_v, sem):
    wid = lax.axis_index("s") * NC + lax.axis_index("c")
    per_w = NT // NW

    def body(cc, carry):
        base = wid * per_w + cc * _OCH
        pltpu.sync_copy(dg_hbm.at[pl.ds(base, _OCH)], idx_v)
        pltpu.async_copy(ye_hbm.at[idx_v], rows_v, sem).wait()
        pltpu.sync_copy(rows_v, out_hbm.at[pl.ds(base, _OCH)])
        return carry

    lax.fori_loop(0, per_w // _OCH, body, 0)


def _combine(dst_g, ye):
    return pl.kernel(
        _combine_body,
        out_type=jax.ShapeDtypeStruct((NT, DIM), jnp.float32),
        mesh=_mesh,
        scratch_types=[
            pltpu.VMEM((_OCH,), jnp.int32),
            pltpu.VMEM((_OCH, DIM), jnp.float32),
            pltpu.SemaphoreType.DMA,
        ],
        compiler_params=_sc_params,
    )(dst_g, ye)


def kernel(x, Wg, W1, W2):
    Bx, Tx, D = x.shape
    xf = x.reshape(Bx * Tx, D)
    dst_sc, dst_g, w, xpk, aux = _route(xf, Wg)
    tok_slot, w_slot = _build(dst_sc, w)
    wsl3 = w_slot[:NSLOT].reshape(E, 1, CAP_PAD)
    ye = None
    for g in range(NG):
        xe_g = _gather_x(tok_slot, xpk, g)
        ye = _ffn(xe_g.reshape(E_G, CAP_PAD, PK), W1, W2, wsl3, g, ye)
    out = _combine(dst_g, ye.reshape(NSLOT, D))
    return out.reshape(Bx, Tx, D), aux[0, 0]
